# Initial kernel scaffold; baseline (speedup 1.0000x reference)
import jax
import jax.numpy as jnp
from jax import lax
from jax.experimental import pallas as pl
from jax.experimental.pallas import tpu as pltpu
from jax.experimental.pallas import tpu_sc as plsc

_NC, _NS = 2, 16
_NW = _NC * _NS
_SZ = [16384, 4096, 1024, 256, 64]
_SEQ = 9
_C = [3, 32, 32, 32, 64]
_BS = 16


def _sc_gather(D, B):
    b_per_w = B // _NW
    assert B % _NW == 0 and b_per_w % 8 == 0, (B,)
    ch = None
    for c in range(min(128, b_per_w), 0, -8):
        if b_per_w % c == 0 and c * D * 4 <= 393216:
            ch = c
            break
    assert ch is not None, (B, D)
    n_chunks = b_per_w // ch
    mesh = plsc.VectorSubcoreMesh(core_axis_name="c", subcore_axis_name="s",
                                  num_cores=_NC, num_subcores=_NS)

    def body(table_hbm, idx_hbm, out_hbm, idx_v, rows_v, sem):
        wid = lax.axis_index("s") * _NC + lax.axis_index("c")
        w0 = wid * b_per_w

        def step(ci, carry):
            base = w0 + ci * ch
            pltpu.sync_copy(idx_hbm.at[pl.ds(base, ch)], idx_v)
            pltpu.async_copy(table_hbm.at[idx_v], rows_v, sem).wait()
            pltpu.sync_copy(rows_v, out_hbm.at[pl.ds(base, ch)])
            return carry

        if n_chunks == 1:
            step(0, 0)
        else:
            lax.fori_loop(0, n_chunks, step, 0)

    return pl.kernel(body,
                     out_type=jax.ShapeDtypeStruct((B, D), jnp.float32),
                     mesh=mesh,
                     scratch_types=[pltpu.VMEM((ch,), jnp.int32),
                                    pltpu.VMEM((ch, D), jnp.float32),
                                    pltpu.SemaphoreType.DMA])


def _conv_tc(G, Ws, brow, elu):
    S, R, Cin = G.shape
    Cout = Ws.shape[2]
    T = min(2048, R)

    def body(g_ref, w_ref, b_ref, o_ref):
        acc = jnp.dot(g_ref[0], w_ref[0], preferred_element_type=jnp.float32)
        for s in range(1, S):
            acc = acc + jnp.dot(g_ref[s], w_ref[s],
                                preferred_element_type=jnp.float32)
        acc = acc + b_ref[...]
        if elu:
            acc = jnp.where(acc > 0, acc, jnp.expm1(acc))
        o_ref[...] = acc

    return pl.pallas_call(
        body,
        grid=(R // T,),
        in_specs=[pl.BlockSpec((S, T, Cin), lambda i: (0, i, 0)),
                  pl.BlockSpec((S, Cin, Cout), lambda i: (0, 0, 0)),
                  pl.BlockSpec((1, Cout), lambda i: (0, 0))],
        out_specs=pl.BlockSpec((T, Cout), lambda i: (i, 0)),
        out_shape=jax.ShapeDtypeStruct((R, Cout), jnp.float32),
    )(G, Ws, brow)


def _pool_tc(G4, val4):
    _, N, D = G4.shape
    T = min(512 if D <= 512 else 256, N)

    def body(g_ref, v_ref, o_ref):
        acc = g_ref[0] * v_ref[0]
        for k in range(1, 4):
            acc = acc + g_ref[k] * v_ref[k]
        o_ref[...] = acc

    return pl.pallas_call(
        body,
        grid=(N // T,),
        in_specs=[pl.BlockSpec((4, T, D), lambda i: (0, i, 0)),
                  pl.BlockSpec((4, T, 1), lambda i: (0, i, 0))],
        out_specs=pl.BlockSpec((T, D), lambda i: (i, 0)),
        out_shape=jax.ShapeDtypeStruct((N, D), jnp.float32),
    )(G4, val4)


def _fc_tc(h, w1t, b1, w2t, b2):
    def body(h_ref, w1_ref, b1_ref, w2_ref, b2_ref, o_ref):
        mu = jnp.dot(h_ref[...], w1_ref[...],
                     preferred_element_type=jnp.float32) + b1_ref[...]
        mu = jax.nn.sigmoid(mu)
        o_ref[...] = jnp.dot(mu, w2_ref[...],
                             preferred_element_type=jnp.float32) + b2_ref[...]

    return pl.pallas_call(
        body,
        out_shape=jax.ShapeDtypeStruct((h.shape[0], w2t.shape[1]),
                                       jnp.float32),
    )(h, w1t, b1, w2t, b2)


def _wstack(W, Cin, Cout):
    return W.reshape(Cout, _SEQ, Cin).transpose(1, 2, 0)


def kernel(x, sp_idx_0, sp_idx_1, sp_idx_2, sp_idx_3,
           dn_row_0, dn_row_1, dn_row_2, dn_row_3,
           dn_col_0, dn_col_1, dn_col_2, dn_col_3,
           dn_val_0, dn_val_1, dn_val_2, dn_val_3,
           up_row_0, up_row_1, up_row_2, up_row_3,
           up_col_0, up_col_1, up_col_2, up_col_3,
           up_val_0, up_val_1, up_val_2, up_val_3,
           enW0, enW1, enW2, enW3, enb0, enb1, enb2, enb3,
           en_fcW, en_fcb, de_fcW, de_fcb,
           deW0, deW1, deW2, deW3, deb0, deb1, deb2, deb3,
           outW, outb):
    sp = [sp_idx_0, sp_idx_1, sp_idx_2, sp_idx_3]
    enW = [enW0, enW1, enW2, enW3]
    enb = [enb0, enb1, enb2, enb3]
    deW = [deW0, deW1, deW2, deW3]
    deb = [deb0, deb1, deb2, deb3]
    dncol = [dn_col_0, dn_col_1, dn_col_2, dn_col_3]
    dnval = [dn_val_0, dn_val_1, dn_val_2, dn_val_3]
    upcol = [up_col_0, up_col_1, up_col_2, up_col_3]
    upval = [up_val_0, up_val_1, up_val_2, up_val_3]

    h = jnp.transpose(x, (1, 0, 2)).reshape(_SZ[0], _BS * _C[0])

    for i in range(4):
        N, Cin, Cout = _SZ[i], _C[i], _C[i + 1]
        D = _BS * Cin
        idxT = jnp.transpose(sp[i]).reshape(-1)
        G = _sc_gather(D, _SEQ * N)(h, idxT)
        conv = _conv_tc(G.reshape(_SEQ, N * _BS, Cin),
                        _wstack(enW[i], Cin, Cout),
                        enb[i].reshape(1, Cout), elu=True)
        conv = conv.reshape(N, _BS * Cout)
        M = _SZ[i + 1]
        colT = jnp.transpose(dncol[i].reshape(M, 4)).reshape(-1)
        val4 = jnp.transpose(dnval[i].reshape(M, 4)).reshape(4, M, 1)
        G4 = _sc_gather(_BS * Cout, 4 * M)(conv, colT)
        h = _pool_tc(G4.reshape(4, M, _BS * Cout), val4)

    hflat = jnp.transpose(h.reshape(_SZ[4], _BS, _C[4]),
                          (1, 0, 2)).reshape(_BS, _SZ[4] * _C[4])
    z = _fc_tc(hflat, en_fcW.T, en_fcb.reshape(1, -1),
               de_fcW.T, de_fcb.reshape(1, -1))
    h = jnp.transpose(z.reshape(_BS, _SZ[4], _C[4]),
                      (1, 0, 2)).reshape(_SZ[4], _BS * _C[4])

    dec_cin = [64, 64, 32, 32]
    dec_cout = [64, 32, 32, 32]
    for j in range(4):
        lvl = 3 - j
        N, M = _SZ[lvl], _SZ[lvl + 1]
        Cin, Cout = dec_cin[j], dec_cout[j]
        D = _BS * Cin
        colT = jnp.transpose(upcol[lvl].reshape(N, 4)).reshape(-1)
        val4 = jnp.transpose(upval[lvl].reshape(N, 4)).reshape(4, N, 1)
        G4 = _sc_gather(D, 4 * N)(h, colT)
        hp = _pool_tc(G4.reshape(4, N, D), val4)
        idxT = jnp.transpose(sp[lvl]).reshape(-1)
        G = _sc_gather(D, _SEQ * N)(hp, idxT)
        conv = _conv_tc(G.reshape(_SEQ, N * _BS, Cin),
                        _wstack(deW[j], Cin, Cout),
                        deb[j].reshape(1, Cout), elu=True)
        h = conv.reshape(N, _BS * Cout)

    idxT = jnp.transpose(sp[0]).reshape(-1)
    G = _sc_gather(_BS * 32, _SEQ * _SZ[0])(h, idxT)
    conv = _conv_tc(G.reshape(_SEQ, _SZ[0] * _BS, 32),
                    _wstack(outW, 32, 3), outb.reshape(1, 3), elu=False)
    return jnp.transpose(conv.reshape(_SZ[0], _BS, 3), (1, 0, 2))


# speedup vs baseline: 5.8032x; 5.8032x over previous
import jax
import jax.numpy as jnp
from jax import lax
from jax.experimental import pallas as pl
from jax.experimental.pallas import tpu as pltpu
from jax.experimental.pallas import tpu_sc as plsc

_NC, _NS = 2, 16
_NW = _NC * _NS
_SZ = [16384, 4096, 1024, 256, 64]
_SEQ = 9
_C = [3, 32, 32, 32, 64]
_BS = 16


def _sc_gather(D, B):
    b_per_w = B // _NW
    assert B % _NW == 0 and b_per_w % 8 == 0, (B,)
    ch = None
    for c in range(min(128, b_per_w), 0, -8):
        if b_per_w % c == 0 and c * D * 4 <= 393216:
            ch = c
            break
    assert ch is not None, (B, D)
    n_chunks = b_per_w // ch
    mesh = plsc.VectorSubcoreMesh(core_axis_name="c", subcore_axis_name="s",
                                  num_cores=_NC, num_subcores=_NS)

    def body(table_hbm, idx_hbm, out_hbm, idx_v, rows_v, sem):
        wid = lax.axis_index("s") * _NC + lax.axis_index("c")
        w0 = wid * b_per_w

        def step(ci, carry):
            base = w0 + ci * ch
            pltpu.sync_copy(idx_hbm.at[pl.ds(base, ch)], idx_v)
            pltpu.async_copy(table_hbm.at[idx_v], rows_v, sem).wait()
            pltpu.sync_copy(rows_v, out_hbm.at[pl.ds(base, ch)])
            return carry

        if n_chunks == 1:
            step(0, 0)
        else:
            lax.fori_loop(0, n_chunks, step, 0)

    return pl.kernel(body,
                     out_type=jax.ShapeDtypeStruct((B, D), jnp.float32),
                     mesh=mesh,
                     scratch_types=[pltpu.VMEM((ch,), jnp.int32),
                                    pltpu.VMEM((ch, D), jnp.float32),
                                    pltpu.SemaphoreType.DMA])


def _conv_tc(G, Ws, brow, elu):
    S, R, Cin = G.shape
    Cout = Ws.shape[2]
    T = min(2048, R)

    def body(g_ref, w_ref, b_ref, o_ref):
        acc = jnp.dot(g_ref[0], w_ref[0], preferred_element_type=jnp.float32)
        for s in range(1, S):
            acc = acc + jnp.dot(g_ref[s], w_ref[s],
                                preferred_element_type=jnp.float32)
        acc = acc + b_ref[...]
        if elu:
            acc = jnp.where(acc > 0, acc, jnp.exp(jnp.minimum(acc, 0.0)) - 1.0)
        o_ref[...] = acc

    return pl.pallas_call(
        body,
        grid=(R // T,),
        in_specs=[pl.BlockSpec((S, T, Cin), lambda i: (0, i, 0)),
                  pl.BlockSpec((S, Cin, Cout), lambda i: (0, 0, 0)),
                  pl.BlockSpec((1, Cout), lambda i: (0, 0))],
        out_specs=pl.BlockSpec((T, Cout), lambda i: (i, 0)),
        out_shape=jax.ShapeDtypeStruct((R, Cout), jnp.float32),
    )(G, Ws, brow)


def _pool_tc(G4, val4):
    _, N, D = G4.shape
    T = min(512 if D <= 512 else 256, N)

    def body(g_ref, v_ref, o_ref):
        acc = g_ref[0] * v_ref[0]
        for k in range(1, 4):
            acc = acc + g_ref[k] * v_ref[k]
        o_ref[...] = acc

    return pl.pallas_call(
        body,
        grid=(N // T,),
        in_specs=[pl.BlockSpec((4, T, D), lambda i: (0, i, 0)),
                  pl.BlockSpec((4, T, 1), lambda i: (0, i, 0))],
        out_specs=pl.BlockSpec((T, D), lambda i: (i, 0)),
        out_shape=jax.ShapeDtypeStruct((N, D), jnp.float32),
    )(G4, val4)


def _fc_tc(h, w1t, b1, w2t, b2):
    def body(h_ref, w1_ref, b1_ref, w2_ref, b2_ref, o_ref):
        mu = jnp.dot(h_ref[...], w1_ref[...],
                     preferred_element_type=jnp.float32) + b1_ref[...]
        mu = jax.nn.sigmoid(mu)
        o_ref[...] = jnp.dot(mu, w2_ref[...],
                             preferred_element_type=jnp.float32) + b2_ref[...]

    return pl.pallas_call(
        body,
        out_shape=jax.ShapeDtypeStruct((h.shape[0], w2t.shape[1]),
                                       jnp.float32),
    )(h, w1t, b1, w2t, b2)


def _wstack(W, Cin, Cout):
    return W.reshape(Cout, _SEQ, Cin).transpose(1, 2, 0)


def kernel(x, sp_idx_0, sp_idx_1, sp_idx_2, sp_idx_3,
           dn_row_0, dn_row_1, dn_row_2, dn_row_3,
           dn_col_0, dn_col_1, dn_col_2, dn_col_3,
           dn_val_0, dn_val_1, dn_val_2, dn_val_3,
           up_row_0, up_row_1, up_row_2, up_row_3,
           up_col_0, up_col_1, up_col_2, up_col_3,
           up_val_0, up_val_1, up_val_2, up_val_3,
           enW0, enW1, enW2, enW3, enb0, enb1, enb2, enb3,
           en_fcW, en_fcb, de_fcW, de_fcb,
           deW0, deW1, deW2, deW3, deb0, deb1, deb2, deb3,
           outW, outb):
    sp = [sp_idx_0, sp_idx_1, sp_idx_2, sp_idx_3]
    enW = [enW0, enW1, enW2, enW3]
    enb = [enb0, enb1, enb2, enb3]
    deW = [deW0, deW1, deW2, deW3]
    deb = [deb0, deb1, deb2, deb3]
    dncol = [dn_col_0, dn_col_1, dn_col_2, dn_col_3]
    dnval = [dn_val_0, dn_val_1, dn_val_2, dn_val_3]
    upcol = [up_col_0, up_col_1, up_col_2, up_col_3]
    upval = [up_val_0, up_val_1, up_val_2, up_val_3]

    h = jnp.transpose(x, (1, 0, 2)).reshape(_SZ[0], _BS * _C[0])
    h = jnp.pad(h, ((0, 0), (0, 128 - _BS * _C[0])))

    for i in range(4):
        N, Cin, Cout = _SZ[i], _C[i], _C[i + 1]
        idxT = jnp.transpose(sp[i]).reshape(-1)
        if i == 0:
            G = _sc_gather(128, _SEQ * N)(h, idxT)
            Wbd = jax.vmap(lambda w: jnp.kron(jnp.eye(_BS, dtype=w.dtype), w)
                           )(_wstack(enW[i], Cin, Cout))
            Wbd = jnp.pad(Wbd, ((0, 0), (0, 128 - _BS * Cin), (0, 0)))
            conv = _conv_tc(G.reshape(_SEQ, N, 128), Wbd,
                            jnp.tile(enb[i], _BS).reshape(1, _BS * Cout),
                            elu=True)
        else:
            D = _BS * Cin
            G = _sc_gather(D, _SEQ * N)(h, idxT)
            conv = _conv_tc(G.reshape(_SEQ, N * _BS, Cin),
                            _wstack(enW[i], Cin, Cout),
                            enb[i].reshape(1, Cout), elu=True)
            conv = conv.reshape(N, _BS * Cout)
        M = _SZ[i + 1]
        colT = jnp.transpose(dncol[i].reshape(M, 4)).reshape(-1)
        val4 = jnp.transpose(dnval[i].reshape(M, 4)).reshape(4, M, 1)
        G4 = _sc_gather(_BS * Cout, 4 * M)(conv, colT)
        h = _pool_tc(G4.reshape(4, M, _BS * Cout), val4)

    hflat = jnp.transpose(h.reshape(_SZ[4], _BS, _C[4]),
                          (1, 0, 2)).reshape(_BS, _SZ[4] * _C[4])
    z = _fc_tc(hflat, en_fcW.T, en_fcb.reshape(1, -1),
               de_fcW.T, de_fcb.reshape(1, -1))
    h = jnp.transpose(z.reshape(_BS, _SZ[4], _C[4]),
                      (1, 0, 2)).reshape(_SZ[4], _BS * _C[4])

    dec_cin = [64, 64, 32, 32]
    dec_cout = [64, 32, 32, 32]
    for j in range(4):
        lvl = 3 - j
        N, M = _SZ[lvl], _SZ[lvl + 1]
        Cin, Cout = dec_cin[j], dec_cout[j]
        D = _BS * Cin
        colT = jnp.transpose(upcol[lvl].reshape(N, 4)).reshape(-1)
        val4 = jnp.transpose(upval[lvl].reshape(N, 4)).reshape(4, N, 1)
        G4 = _sc_gather(D, 4 * N)(h, colT)
        hp = _pool_tc(G4.reshape(4, N, D), val4)
        idxT = jnp.transpose(sp[lvl]).reshape(-1)
        G = _sc_gather(D, _SEQ * N)(hp, idxT)
        conv = _conv_tc(G.reshape(_SEQ, N * _BS, Cin),
                        _wstack(deW[j], Cin, Cout),
                        deb[j].reshape(1, Cout), elu=True)
        h = conv.reshape(N, _BS * Cout)

    idxT = jnp.transpose(sp[0]).reshape(-1)
    G = _sc_gather(_BS * 32, _SEQ * _SZ[0])(h, idxT)
    conv = _conv_tc(G.reshape(_SEQ, _SZ[0] * _BS, 32),
                    _wstack(outW, 32, 3), outb.reshape(1, 3), elu=False)
    return jnp.transpose(conv.reshape(_SZ[0], _BS, 3), (1, 0, 2))
